# Initial kernel scaffold; baseline (speedup 1.0000x reference)
import jax
import jax.numpy as jnp
from jax.experimental import pallas as pl


def kernel(x_tab, x_gnn, edge_index, edge_attr, params):
    raise NotImplementedError("write your pallas kernel here")


# speedup vs baseline: 1.0005x; 1.0005x over previous
import functools
import math

import jax
import jax.numpy as jnp
from jax.experimental import pallas as pl
from jax.experimental.pallas import tpu as pltpu

C = 128
H = 8
DH = C // H
AVG_LOG = math.log(33.0)


def _layer_norm(x, g, b, eps=1e-5):
    m = jnp.mean(x, axis=-1, keepdims=True)
    v = jnp.mean((x - m) ** 2, axis=-1, keepdims=True)
    return (x - m) / jnp.sqrt(v + eps) * g + b


def _bn_combine_kernel(out_ref, xg_ref, bm_ref, bv_ref, g_ref, b_ref, o_ref):
    out = out_ref[...]
    xg = xg_ref[...]
    bm = bm_ref[...]
    bv = bv_ref[...]
    y = (out - bm) / jnp.sqrt(bv + 1e-5) * g_ref[...] + b_ref[...]
    o_ref[...] = (xg + jnp.maximum(y, 0.0)) / 2.0


def kernel(x_tab, x_gnn, edge_index, edge_attr, params):
    p = params
    B, S, _ = x_tab.shape
    q = (x_tab @ p['Wq'] + p['bq']).reshape(B, S, H, DH).transpose(0, 2, 1, 3)
    k = (x_tab @ p['Wk'] + p['bk']).reshape(B, S, H, DH).transpose(0, 2, 1, 3)
    v = (x_tab @ p['Wv'] + p['bv']).reshape(B, S, H, DH).transpose(0, 2, 1, 3)
    attn = jax.nn.softmax(q @ k.transpose(0, 1, 3, 2) / jnp.sqrt(float(DH)), axis=-1)
    ctx = (attn @ v).transpose(0, 2, 1, 3).reshape(B, S, C) @ p['Wo'] + p['bo']
    x1 = _layer_norm(x_tab + ctx, p['ln1_g'], p['ln1_b'])
    ff = jax.nn.relu(x1 @ p['W1'] + p['b1']) @ p['W2'] + p['b2']
    x2 = _layer_norm(x1 + ff, p['ln2_g'], p['ln2_b'])
    x_tab_out = _layer_norm(x2, p['tn_g'], p['tn_b'])

    src = edge_index[0]
    dst = edge_index[1]
    N = x_gnn.shape[0]
    e = edge_attr @ p['We'] + p['be']
    h = jnp.concatenate([x_gnn[dst], x_gnn[src], e], axis=-1) @ p['Wpre'] + p['bpre']
    cnt = jax.ops.segment_sum(jnp.ones((h.shape[0],), jnp.float32), dst, num_segments=N)
    cnt1 = jnp.maximum(cnt, 1.0)
    s = jax.ops.segment_sum(h, dst, num_segments=N)
    mean = s / cnt1[:, None]
    mx = jax.ops.segment_max(h, dst, num_segments=N)
    mx = jnp.where(cnt[:, None] > 0, mx, 0.0)
    mn = jax.ops.segment_min(h, dst, num_segments=N)
    mn = jnp.where(cnt[:, None] > 0, mn, 0.0)
    mean_sq = jax.ops.segment_sum(h * h, dst, num_segments=N) / cnt1[:, None]
    var = mean_sq - mean * mean
    std = jnp.sqrt(jax.nn.relu(var) + 1e-5)
    agg = jnp.concatenate([mean, mx, mn, std], axis=-1)
    amp = (jnp.log(cnt1 + 1.0) / AVG_LOG)[:, None]
    att = (AVG_LOG / jnp.log(cnt1 + 1.0))[:, None]
    sc = jnp.concatenate([agg, agg * amp, agg * att], axis=-1)
    out = jnp.concatenate([x_gnn, sc], axis=-1) @ p['Wpost'] + p['bpost']
    out = out @ p['Wlin'] + p['blin']
    bm = jnp.mean(out, axis=0)
    bv = jnp.var(out, axis=0)

    x_gnn_out = pl.pallas_call(
        _bn_combine_kernel,
        out_shape=jax.ShapeDtypeStruct((N, C), jnp.float32),
        grid=(N // 2000,),
        in_specs=[
            pl.BlockSpec((2000, C), lambda i: (i, 0)),
            pl.BlockSpec((2000, C), lambda i: (i, 0)),
            pl.BlockSpec((1, C), lambda i: (0, 0)),
            pl.BlockSpec((1, C), lambda i: (0, 0)),
            pl.BlockSpec((1, C), lambda i: (0, 0)),
            pl.BlockSpec((1, C), lambda i: (0, 0)),
        ],
        out_specs=pl.BlockSpec((2000, C), lambda i: (i, 0)),
    )(out, x_gnn, bm.reshape(1, C), bv.reshape(1, C),
      p['bn_g'].reshape(1, C), p['bn_b'].reshape(1, C))

    return (x_tab_out, x_gnn_out, edge_attr)


# speedup vs baseline: 1.3114x; 1.3107x over previous
import functools
import math

import numpy as np
import jax
import jax.numpy as jnp
from jax.experimental import pallas as pl
from jax.experimental.pallas import tpu as pltpu

C = 128
H = 8
S = 8
DH = C // H
AVG_LOG = math.log(33.0)

_d = np.arange(DH)
_E_big = np.zeros((S * C, H * S), np.float32)
for t in range(S):
    for h in range(H):
        _E_big[t * C + h * DH + _d, h * S + t] = 1.0 / math.sqrt(DH)
_ONES8 = np.kron(np.eye(H), np.ones((S, S))).astype(np.float32)
_F = np.zeros((H * S, S * C), np.float32)
for t in range(S):
    for h in range(H):
        _F[h * S + t, t * C + h * DH + _d] = 1.0
_G = np.tile(np.eye(C, dtype=np.float32), (S, 1))


def _ln(x, g, b, eps=1e-5):
    m = jnp.mean(x, axis=-1, keepdims=True)
    xc = x - m
    v = jnp.mean(xc * xc, axis=-1, keepdims=True)
    return xc * jax.lax.rsqrt(v + eps) * g + b


def _tab_kernel(x_ref, wq, bq, wk, bk, wv, bv, wo, bo, w1, b1, w2, b2,
                ln1g, ln1b, ln2g, ln2b, tng, tnb, ebig, ones8, fmat, gmat,
                o_ref):
    q = []
    k = []
    vcat_parts = []
    for s_i in range(S):
        xs = x_ref[s_i]
        q.append(jnp.dot(xs, wq[...], preferred_element_type=jnp.float32) + bq[...])
        k.append(jnp.dot(xs, wk[...], preferred_element_type=jnp.float32) + bk[...])
        vcat_parts.append(
            jnp.dot(xs, wv[...], preferred_element_type=jnp.float32) + bv[...])
    vcat = jnp.concatenate(vcat_parts, axis=-1)
    for s_i in range(S):
        qk = jnp.concatenate([q[s_i] * k[t] for t in range(S)], axis=-1)
        scores = jnp.dot(qk, ebig[...], preferred_element_type=jnp.float32)
        es = jnp.exp(scores)
        denom = jnp.dot(es, ones8[...], preferred_element_type=jnp.float32)
        attn = es / denom
        bcast = jnp.dot(attn, fmat[...], preferred_element_type=jnp.float32)
        ctx = jnp.dot(bcast * vcat, gmat[...], preferred_element_type=jnp.float32)
        proj = jnp.dot(ctx, wo[...], preferred_element_type=jnp.float32) + bo[...]
        x1 = _ln(x_ref[s_i] + proj, ln1g[...], ln1b[...])
        ff = jnp.maximum(
            jnp.dot(x1, w1[...], preferred_element_type=jnp.float32) + b1[...], 0.0)
        ff = jnp.dot(ff, w2[...], preferred_element_type=jnp.float32) + b2[...]
        x2 = _ln(x1 + ff, ln2g[...], ln2b[...])
        o_ref[s_i] = _ln(x2, tng[...], tnb[...])


def _tab_branch(x_tab_t, p, B):
    N = x_tab_t.shape[1]
    row = lambda a: a.reshape(1, C)
    w_spec = lambda shp: pl.BlockSpec(shp, lambda i: (0, 0))
    args = [
        p['Wq'], row(p['bq']), p['Wk'], row(p['bk']), p['Wv'], row(p['bv']),
        p['Wo'], row(p['bo']), p['W1'], row(p['b1']), p['W2'], row(p['b2']),
        row(p['ln1_g']), row(p['ln1_b']), row(p['ln2_g']), row(p['ln2_b']),
        row(p['tn_g']), row(p['tn_b']),
        jnp.asarray(_E_big), jnp.asarray(_ONES8), jnp.asarray(_F), jnp.asarray(_G),
    ]
    specs = [w_spec(a.shape) for a in args]
    return pl.pallas_call(
        _tab_kernel,
        out_shape=jax.ShapeDtypeStruct((S, N, C), jnp.float32),
        grid=(N // B,),
        in_specs=[pl.BlockSpec((S, B, C), lambda i: (0, i, 0))] + specs,
        out_specs=pl.BlockSpec((S, B, C), lambda i: (0, i, 0)),
    )(x_tab_t, *args)


def _bn_combine_kernel(out_ref, xg_ref, bm_ref, bv_ref, g_ref, b_ref, o_ref):
    y = (out_ref[...] - bm_ref[...]) / jnp.sqrt(bv_ref[...] + 1e-5) \
        * g_ref[...] + b_ref[...]
    o_ref[...] = (xg_ref[...] + jnp.maximum(y, 0.0)) / 2.0


def kernel(x_tab, x_gnn, edge_index, edge_attr, params):
    p = params
    N = x_gnn.shape[0]

    x_tab_t = x_tab.transpose(1, 0, 2)
    x_tab_out = _tab_branch(x_tab_t, p, 1000).transpose(1, 0, 2)

    src = edge_index[0]
    dst = edge_index[1]
    Wp1, Wp2, Wp3 = p['Wpre'][:C], p['Wpre'][C:2 * C], p['Wpre'][2 * C:]
    a = x_gnn @ Wp1
    b = x_gnn @ Wp2
    c = edge_attr @ (p['We'] @ Wp3) + (p['be'] @ Wp3 + p['bpre'])
    r = b[src] + c
    cnt = jax.ops.segment_sum(jnp.ones((r.shape[0],), jnp.float32), dst,
                              num_segments=N)
    cnt1 = jnp.maximum(cnt, 1.0)
    S1 = jax.ops.segment_sum(r, dst, num_segments=N)
    S2 = jax.ops.segment_sum(r * r, dst, num_segments=N)
    M = jax.ops.segment_max(r, dst, num_segments=N)
    m = jax.ops.segment_min(r, dst, num_segments=N)

    pos = cnt[:, None] > 0
    mean = jnp.where(pos, a + S1 / cnt1[:, None], 0.0)
    mx = jnp.where(pos, a + M, 0.0)
    mn = jnp.where(pos, a + m, 0.0)
    var = S2 / cnt1[:, None] - (S1 / cnt1[:, None]) ** 2
    std = jnp.sqrt(jax.nn.relu(var) + 1e-5)
    agg = jnp.concatenate([mean, mx, mn, std], axis=-1)
    amp = (jnp.log(cnt1 + 1.0) / AVG_LOG)[:, None]
    att = (AVG_LOG / jnp.log(cnt1 + 1.0))[:, None]
    sc = jnp.concatenate([agg, agg * amp, agg * att], axis=-1)
    out = jnp.concatenate([x_gnn, sc], axis=-1) @ p['Wpost'] + p['bpost']
    out = out @ p['Wlin'] + p['blin']
    bm = jnp.mean(out, axis=0)
    bv = jnp.var(out, axis=0)

    x_gnn_out = pl.pallas_call(
        _bn_combine_kernel,
        out_shape=jax.ShapeDtypeStruct((N, C), jnp.float32),
        grid=(N // 2000,),
        in_specs=[
            pl.BlockSpec((2000, C), lambda i: (i, 0)),
            pl.BlockSpec((2000, C), lambda i: (i, 0)),
            pl.BlockSpec((1, C), lambda i: (0, 0)),
            pl.BlockSpec((1, C), lambda i: (0, 0)),
            pl.BlockSpec((1, C), lambda i: (0, 0)),
            pl.BlockSpec((1, C), lambda i: (0, 0)),
        ],
        out_specs=pl.BlockSpec((2000, C), lambda i: (i, 0)),
    )(out, x_gnn, bm.reshape(1, C), bv.reshape(1, C),
      p['bn_g'].reshape(1, C), p['bn_b'].reshape(1, C))

    return (x_tab_out, x_gnn_out, edge_attr)


# speedup vs baseline: 1.4871x; 1.1340x over previous
import functools
import math

import numpy as np
import jax
import jax.numpy as jnp
from jax import lax
from jax.experimental import pallas as pl
from jax.experimental.pallas import tpu as pltpu
from jax.experimental.pallas import tpu_sc as plsc

C = 128
H = 8
S = 8
DH = C // H
AVG_LOG = math.log(33.0)

E = 320000
N = 10000
NP = 10016
OSZ = NP // 8
ACC_R = OSZ + 4
SLICES = 32
SLICE_E = E // SLICES
CH = 128
SCAN_CH = 2000
CAP_T = E + SLICES * 8 * CH
NEG = -3.0e38
POS = 3.0e38

_d = np.arange(DH)
_E_big = np.zeros((S * C, H * S), np.float32)
_F = np.zeros((H * S, S * C), np.float32)
for _t in range(S):
    for _h in range(H):
        _E_big[_t * C + _h * DH + _d, _h * S + _t] = 1.0 / math.sqrt(DH)
        _F[_h * S + _t, _t * C + _h * DH + _d] = 1.0
_ONES8 = np.kron(np.eye(H), np.ones((S, S))).astype(np.float32)
_G = np.tile(np.eye(C, dtype=np.float32), (S, 1))


def _ln(x, g, b, eps=1e-5):
    m = jnp.mean(x, axis=-1, keepdims=True)
    xc = x - m
    v = jnp.mean(xc * xc, axis=-1, keepdims=True)
    return xc * lax.rsqrt(v + eps) * g + b


def _tab_kernel(x_ref, wq, bq, wk, bk, wv, bv, wo, bo, w1, b1, w2, b2,
                ln1g, ln1b, ln2g, ln2b, tng, tnb, ebig, ones8, fmat, gmat,
                o_ref):
    q = []
    k = []
    vcat_parts = []
    for s_i in range(S):
        xs = x_ref[s_i]
        q.append(jnp.dot(xs, wq[...], preferred_element_type=jnp.float32) + bq[...])
        k.append(jnp.dot(xs, wk[...], preferred_element_type=jnp.float32) + bk[...])
        vcat_parts.append(
            jnp.dot(xs, wv[...], preferred_element_type=jnp.float32) + bv[...])
    vcat = jnp.concatenate(vcat_parts, axis=-1)
    for s_i in range(S):
        qk = jnp.concatenate([q[s_i] * k[t] for t in range(S)], axis=-1)
        scores = jnp.dot(qk, ebig[...], preferred_element_type=jnp.float32)
        es = jnp.exp(scores)
        denom = jnp.dot(es, ones8[...], preferred_element_type=jnp.float32)
        attn = es / denom
        bcast = jnp.dot(attn, fmat[...], preferred_element_type=jnp.float32)
        ctx = jnp.dot(bcast * vcat, gmat[...], preferred_element_type=jnp.float32)
        proj = jnp.dot(ctx, wo[...], preferred_element_type=jnp.float32) + bo[...]
        x1 = _ln(x_ref[s_i] + proj, ln1g[...], ln1b[...])
        ff = jnp.maximum(
            jnp.dot(x1, w1[...], preferred_element_type=jnp.float32) + b1[...], 0.0)
        ff = jnp.dot(ff, w2[...], preferred_element_type=jnp.float32) + b2[...]
        x2 = _ln(x1 + ff, ln2g[...], ln2b[...])
        o_ref[s_i] = _ln(x2, tng[...], tnb[...])


def _tab_branch(x_tab_t, p, B):
    n = x_tab_t.shape[1]
    row = lambda a: a.reshape(1, C)
    args = [
        p['Wq'], row(p['bq']), p['Wk'], row(p['bk']), p['Wv'], row(p['bv']),
        p['Wo'], row(p['bo']), p['W1'], row(p['b1']), p['W2'], row(p['b2']),
        row(p['ln1_g']), row(p['ln1_b']), row(p['ln2_g']), row(p['ln2_b']),
        row(p['tn_g']), row(p['tn_b']),
        jnp.asarray(_E_big), jnp.asarray(_ONES8), jnp.asarray(_F), jnp.asarray(_G),
    ]
    specs = [pl.BlockSpec(a.shape, lambda i: (0, 0)) for a in args]
    return pl.pallas_call(
        _tab_kernel,
        out_shape=jax.ShapeDtypeStruct((S, n, C), jnp.float32),
        grid=(n // B,),
        in_specs=[pl.BlockSpec((S, B, C), lambda i: (0, i, 0))] + specs,
        out_specs=pl.BlockSpec((S, B, C), lambda i: (0, i, 0)),
    )(x_tab_t, *args)


def _mm_bias_kernel(x_ref, w_ref, b_ref, o_ref):
    o_ref[...] = jnp.dot(x_ref[...], w_ref[...],
                         preferred_element_type=jnp.float32) + b_ref[...]


def _mm_bias(x, w, bias, blk):
    n = x.shape[0]
    return pl.pallas_call(
        _mm_bias_kernel,
        out_shape=jax.ShapeDtypeStruct((n, C), jnp.float32),
        grid=(n // blk,),
        in_specs=[
            pl.BlockSpec((blk, x.shape[1]), lambda i: (i, 0)),
            pl.BlockSpec(w.shape, lambda i: (0, 0)),
            pl.BlockSpec((1, C), lambda i: (0, 0)),
        ],
        out_specs=pl.BlockSpec((blk, C), lambda i: (i, 0)),
    )(x, w, bias.reshape(1, C))


def _ab_kernel(x_ref, w1_ref, w2_ref, a_ref, b_ref):
    x = x_ref[...]
    a_ref[...] = jnp.dot(x, w1_ref[...], preferred_element_type=jnp.float32)
    b_ref[...] = jnp.dot(x, w2_ref[...], preferred_element_type=jnp.float32)


def _ab_proj(x_pad, wp1, wp2):
    blk = NP // 4
    return pl.pallas_call(
        _ab_kernel,
        out_shape=[jax.ShapeDtypeStruct((NP, C), jnp.float32)] * 2,
        grid=(4,),
        in_specs=[
            pl.BlockSpec((blk, C), lambda i: (i, 0)),
            pl.BlockSpec((C, C), lambda i: (0, 0)),
            pl.BlockSpec((C, C), lambda i: (0, 0)),
        ],
        out_specs=[pl.BlockSpec((blk, C), lambda i: (i, 0))] * 2,
    )(x_pad, wp1, wp2)


_SC_PARAMS = pltpu.CompilerParams(needs_layout_passes=False,
                                  use_tc_tiling_on_sc=False)


def _sc_mesh():
    return plsc.VectorSubcoreMesh(core_axis_name="c", subcore_axis_name="s")


def _k1_count(dst):
    @functools.partial(
        pl.kernel, mesh=_sc_mesh(), compiler_params=_SC_PARAMS,
        out_type=jax.ShapeDtypeStruct((SLICES, 16), jnp.int32),
        scratch_types=[
            pltpu.VMEM((SCAN_CH,), jnp.int32),
            pltpu.VMEM((16,), jnp.int32),
        ],
    )
    def k(dst_hbm, cnt_hbm, dbuf, cvec_v):
        wid = lax.axis_index("s") * 2 + lax.axis_index("c")
        iot = lax.iota(jnp.int32, 16)
        cvec = jnp.zeros((16,), jnp.int32)
        for ch in range(SLICE_E // SCAN_CH):
            pltpu.sync_copy(
                dst_hbm.at[pl.ds(wid * SLICE_E + ch * SCAN_CH, SCAN_CH)], dbuf)

            def body(i, cv):
                d16 = dbuf[pl.ds(i * 16, 16)]
                oct16 = d16 // OSZ
                for o in range(8):
                    pc = plsc.all_reduce_population_count(oct16 == o)
                    cv = cv + jnp.where(iot == o, pc, 0)
                return cv
            cvec = lax.fori_loop(0, SCAN_CH // 16, body, cvec)
        cvec_v[...] = cvec
        pltpu.sync_copy(cvec_v, cnt_hbm.at[wid])
    return k(dst)


def _k2_bucket(dst, src, c, b_pad, bases):
    @functools.partial(
        pl.kernel, mesh=_sc_mesh(), compiler_params=_SC_PARAMS,
        out_type=[
            jax.ShapeDtypeStruct((8, CAP_T, 16), jnp.float32),
            jax.ShapeDtypeStruct((CAP_T,), jnp.int32),
        ],
        scratch_types=[
            pltpu.VMEM((SCAN_CH,), jnp.int32),
            pltpu.VMEM((SCAN_CH,), jnp.int32),
            pltpu.VMEM((8, 256), jnp.int32),
            pltpu.VMEM((8, 256), jnp.int32),
            pltpu.VMEM((8, 256), jnp.int32),
            pltpu.VMEM((CH, C), jnp.float32),
            pltpu.VMEM((CH, C), jnp.float32),
            pltpu.VMEM((8, CH, 16), jnp.float32),
            pltpu.VMEM((16,), jnp.int32),
            pltpu.SMEM((8,), jnp.int32),
            pltpu.SMEM((8,), jnp.int32),
            pltpu.SemaphoreType.DMA,
            pltpu.SemaphoreType.DMA,
        ],
    )
    def k(dst_hbm, src_hbm, c_hbm, b_hbm, bases_hbm, r_hbm, dl_hbm,
          dbuf, sbuf, ids_st, srcs_st, dsts_st, cbuf, bbuf, rg, bvec_v,
          fill_s, wr_s, sem_c, sem_b):
        wid = lax.axis_index("s") * 2 + lax.axis_index("c")
        iot = lax.iota(jnp.int32, 16)
        pltpu.sync_copy(bases_hbm.at[wid], bvec_v)
        bvec = bvec_v[...]
        for o in range(8):
            fill_s[o] = 0
            wr_s[o] = 0

        def flush(o):
            wb = wr_s[o]
            base = bvec[o]
            pos0 = pl.multiple_of(base + wb, CH)
            pltpu.async_copy(c_hbm.at[ids_st.at[o, pl.ds(0, CH)]], cbuf,
                             sem_c).wait()
            pltpu.async_copy(b_hbm.at[srcs_st.at[o, pl.ds(0, CH)]], bbuf,
                             sem_b).wait()

            def comb(j, _):
                for g in range(8):
                    rg[g, j, :] = (cbuf[j, pl.ds(g * 16, 16)]
                                   + bbuf[j, pl.ds(g * 16, 16)])
                return 0
            lax.fori_loop(0, CH, comb, 0)
            for g in range(8):
                pltpu.sync_copy(rg.at[g], r_hbm.at[g, pl.ds(pos0, CH)])
            pltpu.sync_copy(dsts_st.at[o, pl.ds(0, CH)],
                            dl_hbm.at[pl.ds(pos0, CH)])
            wr_s[o] = wb + CH

        def scan_chunk(ch, _):
            ebase = wid * SLICE_E + ch * SCAN_CH
            pltpu.sync_copy(dst_hbm.at[pl.ds(ebase, SCAN_CH)], dbuf)
            pltpu.sync_copy(src_hbm.at[pl.ds(ebase, SCAN_CH)], sbuf)

            def body(i, _):
                d16 = dbuf[pl.ds(i * 16, 16)]
                s16 = sbuf[pl.ds(i * 16, 16)]
                id16 = iot + (ebase + i * 16)
                oct16 = d16 // OSZ
                for o in range(8):
                    msk = oct16 == o
                    pc = plsc.all_reduce_population_count(msk)
                    f = fill_s[o]
                    plsc.store_compressed(ids_st.at[o, pl.ds(f, 16)], id16,
                                          mask=msk)
                    plsc.store_compressed(srcs_st.at[o, pl.ds(f, 16)], s16,
                                          mask=msk)
                    plsc.store_compressed(dsts_st.at[o, pl.ds(f, 16)], d16,
                                          mask=msk)
                    fill_s[o] = f + pc[0]
                for o in range(8):
                    f = fill_s[o]

                    @pl.when(f >= CH)
                    def _():
                        flush(o)
                        t0 = ids_st[o, pl.ds(CH, 16)]
                        t1 = srcs_st[o, pl.ds(CH, 16)]
                        t2 = dsts_st[o, pl.ds(CH, 16)]
                        ids_st[o, pl.ds(0, 16)] = t0
                        srcs_st[o, pl.ds(0, 16)] = t1
                        dsts_st[o, pl.ds(0, 16)] = t2
                        fill_s[o] = f - CH
                return 0
            lax.fori_loop(0, SCAN_CH // 16, body, 0)
            return 0
        lax.fori_loop(0, SLICE_E // SCAN_CH, scan_chunk, 0)

        zid = jnp.zeros((16,), jnp.int32)
        for o in range(8):
            f = fill_s[o]

            @pl.when(f > 0)
            def _():
                sent = jnp.full((16,), o * OSZ + OSZ, jnp.int32)
                for j in range(CH // 16):
                    ids_st[o, pl.ds(f + j * 16, 16)] = zid
                    srcs_st[o, pl.ds(f + j * 16, 16)] = zid
                    dsts_st[o, pl.ds(f + j * 16, 16)] = sent
                flush(o)
    return k(dst, src, c, b_pad, bases)


def _k3_reduce(r_sc, dlist, bases, padded):
    @functools.partial(
        pl.kernel, mesh=_sc_mesh(), compiler_params=_SC_PARAMS,
        out_type=[
            jax.ShapeDtypeStruct((8, 8, ACC_R, 16), jnp.float32),
            jax.ShapeDtypeStruct((8, 8, ACC_R, 16), jnp.float32),
            jax.ShapeDtypeStruct((8, 8, ACC_R, 16), jnp.float32),
            jax.ShapeDtypeStruct((8, 8, ACC_R, 16), jnp.float32),
            jax.ShapeDtypeStruct((8, ACC_R * 16), jnp.float32),
        ],
        scratch_types=[
            pltpu.VMEM((ACC_R, 16), jnp.float32),
            pltpu.VMEM((ACC_R, 16), jnp.float32),
            pltpu.VMEM((ACC_R, 16), jnp.float32),
            pltpu.VMEM((ACC_R, 16), jnp.float32),
            pltpu.VMEM((ACC_R * 16,), jnp.float32),
            pltpu.VMEM((CH, 16), jnp.float32),
            pltpu.VMEM((CH,), jnp.int32),
            pltpu.VMEM((SLICES, 16), jnp.int32),
            pltpu.VMEM((SLICES, 16), jnp.int32),
        ],
    )
    def k(r_hbm, dl_hbm, bases_hbm, pad_hbm, s1_hbm, s2_hbm, mx_hbm, mn_hbm,
          cnt_hbm, a1, a2, aM0, am0, acn, rbuf, dbuf, bas_v, pad_v):
        wid = lax.axis_index("s") * 2 + lax.axis_index("c")
        g = wid % 8
        slot = wid // 8
        iot = lax.iota(jnp.int32, 16)
        ones = jnp.ones((16,), jnp.float32)
        m0 = iot == 0
        pltpu.sync_copy(bases_hbm, bas_v)
        pltpu.sync_copy(pad_hbm, pad_v)
        for rnd in range(2):
            o = slot + 4 * rnd
            obase = o * OSZ

            def init(i, _):
                z = jnp.zeros((16,), jnp.float32)
                a1[i, :] = z
                a2[i, :] = z
                aM0[i, :] = jnp.full((16,), NEG, jnp.float32)
                am0[i, :] = jnp.full((16,), POS, jnp.float32)
                acn[pl.ds(i * 16, 16)] = z
                return 0
            lax.fori_loop(0, ACC_R, init, 0)

            def slice_body(p, _):
                brow = bas_v[p, :]
                prow = pad_v[p, :]
                base = brow[jnp.full((16,), o, jnp.int32)][0]
                npad = prow[jnp.full((16,), o, jnp.int32)][0]

                def chunk(ci, _):
                    off = pl.multiple_of(base + ci * CH, CH)
                    pltpu.sync_copy(r_hbm.at[g, pl.ds(off, CH)], rbuf)
                    pltpu.sync_copy(dl_hbm.at[pl.ds(off, CH)], dbuf)

                    def vstep(v, _):
                        d16 = dbuf[pl.ds(v * 16, 16)] - obase
                        for j in range(16):
                            db = d16[jnp.full((16,), j, jnp.int32)]
                            r = rbuf[v * 16 + j, :]
                            plsc.addupdate_scatter(a1, [db, iot], r)
                            plsc.addupdate_scatter(a2, [db, iot], r * r)
                            om = plsc.load_gather(aM0, [db, iot])
                            plsc.store_scatter(aM0, [db, iot],
                                               jnp.maximum(om, r))
                            on = plsc.load_gather(am0, [db, iot])
                            plsc.store_scatter(am0, [db, iot],
                                               jnp.minimum(on, r))
                            plsc.addupdate_scatter(acn, [db * 16], ones,
                                                   mask=m0)
                        return 0
                    lax.fori_loop(0, CH // 16, vstep, 0)
                    return 0
                lax.fori_loop(0, npad // CH, chunk, 0)
                return 0
            lax.fori_loop(0, SLICES, slice_body, 0)

            pltpu.sync_copy(a1, s1_hbm.at[g, o])
            pltpu.sync_copy(a2, s2_hbm.at[g, o])
            pltpu.sync_copy(aM0, mx_hbm.at[g, o])
            pltpu.sync_copy(am0, mn_hbm.at[g, o])

            @pl.when(g == 0)
            def _():
                pltpu.sync_copy(acn, cnt_hbm.at[o])
    return k(r_sc, dlist, bases, padded)


def _node_kernel(xg_ref, a_ref, cb_ref, s1_ref, s2_ref, mx_ref, mn_ref,
                 wpost_ref, bpost_ref, wlin_ref, blin_ref,
                 out_ref, bs_ref, bq_ref):
    i = pl.program_id(0)
    cnt = cb_ref[...]
    cnt1 = jnp.maximum(cnt, 1.0)
    pos = cnt > 0.0
    inv = 1.0 / cnt1
    a = a_ref[...]
    s1 = s1_ref[...] * inv
    mean = jnp.where(pos, a + s1, 0.0)
    mx = jnp.where(pos, a + mx_ref[...], 0.0)
    mn = jnp.where(pos, a + mn_ref[...], 0.0)
    var = s2_ref[...] * inv - s1 * s1
    std = jnp.sqrt(jnp.maximum(var, 0.0) + 1e-5)
    lg = jnp.log(cnt1 + 1.0)
    amp = lg * (1.0 / AVG_LOG)
    att = AVG_LOG / lg
    wp = wpost_ref
    acc = jnp.dot(xg_ref[...], wp[pl.ds(0, C), :],
                  preferred_element_type=jnp.float32)
    aggs = [mean, mx, mn, std]
    for sc_i, scl in enumerate([None, amp, att]):
        for a_i in range(4):
            blkidx = 1 + sc_i * 4 + a_i
            x = aggs[a_i] if scl is None else aggs[a_i] * scl
            acc = acc + jnp.dot(x, wp[pl.ds(blkidx * C, C), :],
                                preferred_element_type=jnp.float32)
    acc = acc + bpost_ref[...]
    out = jnp.dot(acc, wlin_ref[...],
                  preferred_element_type=jnp.float32) + blin_ref[...]
    out_ref[...] = out

    @pl.when(i == 0)
    def _():
        bs_ref[...] = jnp.zeros_like(bs_ref)
        bq_ref[...] = jnp.zeros_like(bq_ref)
    bs_ref[...] += jnp.sum(out, axis=0, keepdims=True)
    bq_ref[...] += jnp.sum(out * out, axis=0, keepdims=True)


def _node_stage(x_gnn, a, cnt_b, s1, s2, mx, mn, p):
    blk = 2000
    cspec = lambda shp: pl.BlockSpec(shp, lambda i: (0, 0))
    bspec = pl.BlockSpec((blk, C), lambda i: (i, 0))
    return pl.pallas_call(
        _node_kernel,
        out_shape=[
            jax.ShapeDtypeStruct((N, C), jnp.float32),
            jax.ShapeDtypeStruct((1, C), jnp.float32),
            jax.ShapeDtypeStruct((1, C), jnp.float32),
        ],
        grid=(N // blk,),
        in_specs=[bspec] * 7 + [
            cspec((13 * C, C)), cspec((1, C)), cspec((C, C)), cspec((1, C)),
        ],
        out_specs=[bspec, cspec((1, C)), cspec((1, C))],
    )(x_gnn, a, cnt_b, s1, s2, mx, mn,
      p['Wpost'], p['bpost'].reshape(1, C), p['Wlin'], p['blin'].reshape(1, C))


def _bn_combine_kernel(out_ref, xg_ref, bm_ref, bv_ref, g_ref, b_ref, o_ref):
    y = (out_ref[...] - bm_ref[...]) / jnp.sqrt(bv_ref[...] + 1e-5) \
        * g_ref[...] + b_ref[...]
    o_ref[...] = (xg_ref[...] + jnp.maximum(y, 0.0)) / 2.0


def kernel(x_tab, x_gnn, edge_index, edge_attr, params):
    p = params

    x_tab_t = x_tab.transpose(1, 0, 2)
    x_tab_out = _tab_branch(x_tab_t, p, 1000).transpose(1, 0, 2)

    src = edge_index[0]
    dst = edge_index[1]
    Wp1, Wp2, Wp3 = p['Wpre'][:C], p['Wpre'][C:2 * C], p['Wpre'][2 * C:]

    x_pad = jnp.concatenate([x_gnn, jnp.zeros((NP - N, C), jnp.float32)], 0)
    a_pad, b_pad = _ab_proj(x_pad, Wp1, Wp2)
    a = a_pad[:N]
    c = _mm_bias(edge_attr, p['We'] @ Wp3, p['be'] @ Wp3 + p['bpre'], 2000)

    counts = _k1_count(dst)[:, :8]
    padded = ((counts + (CH - 1)) // CH) * CH
    bases_flat = jnp.concatenate(
        [jnp.zeros((1,), jnp.int32), jnp.cumsum(padded.reshape(-1))[:-1]])
    bases = bases_flat.reshape(SLICES, 8)
    bases16 = jnp.concatenate(
        [bases, jnp.zeros((SLICES, 8), jnp.int32)], axis=1)
    padded16 = jnp.concatenate(
        [padded, jnp.zeros((SLICES, 8), jnp.int32)], axis=1)

    r_sc, dlist = _k2_bucket(dst, src, c, b_pad, bases16)
    s1_4, s2_4, mx_4, mn_4, cnt_2 = _k3_reduce(r_sc, dlist, bases16, padded16)

    def _unacc(x4):
        return (x4[:, :, :OSZ, :].transpose(1, 2, 0, 3)
                .reshape(NP, C)[:N])
    s1 = _unacc(s1_4)
    s2 = _unacc(s2_4)
    mxa = _unacc(mx_4)
    mna = _unacc(mn_4)
    cnt = cnt_2.reshape(8, ACC_R, 16)[:, :OSZ, 0].reshape(NP)[:N]
    cnt_b = jnp.broadcast_to(cnt[:, None], (N, C))

    out_pre, bs, bq = _node_stage(x_gnn, a, cnt_b, s1, s2, mxa, mna, p)
    bm = bs / float(N)
    bv = bq / float(N) - bm * bm

    x_gnn_out = pl.pallas_call(
        _bn_combine_kernel,
        out_shape=jax.ShapeDtypeStruct((N, C), jnp.float32),
        grid=(N // 2000,),
        in_specs=[
            pl.BlockSpec((2000, C), lambda i: (i, 0)),
            pl.BlockSpec((2000, C), lambda i: (i, 0)),
            pl.BlockSpec((1, C), lambda i: (0, 0)),
            pl.BlockSpec((1, C), lambda i: (0, 0)),
            pl.BlockSpec((1, C), lambda i: (0, 0)),
            pl.BlockSpec((1, C), lambda i: (0, 0)),
        ],
        out_specs=pl.BlockSpec((2000, C), lambda i: (i, 0)),
    )(out_pre, x_gnn, bm, bv, p['bn_g'].reshape(1, C), p['bn_b'].reshape(1, C))

    return (x_tab_out, x_gnn_out, edge_attr)


# speedup vs baseline: 1.6128x; 1.0845x over previous
import functools
import math

import numpy as np
import jax
import jax.numpy as jnp
from jax import lax
from jax.experimental import pallas as pl
from jax.experimental.pallas import tpu as pltpu
from jax.experimental.pallas import tpu_sc as plsc

C = 128
H = 8
S = 8
DH = C // H
AVG_LOG = math.log(33.0)

E = 320000
N = 10000
NP = 10016
OSZ = NP // 8
ACC_R = OSZ + 4
SLICES = 32
SLICE_E = E // SLICES
CH = 128
SCAN_CH = 2000
CAP_T = E + SLICES * 8 * CH
NEG = -3.0e38
POS = 3.0e38

_d = np.arange(DH)
_E_big = np.zeros((S * C, H * S), np.float32)
_F = np.zeros((H * S, S * C), np.float32)
for _t in range(S):
    for _h in range(H):
        _E_big[_t * C + _h * DH + _d, _h * S + _t] = 1.0 / math.sqrt(DH)
        _F[_h * S + _t, _t * C + _h * DH + _d] = 1.0
_ONES8 = np.kron(np.eye(H), np.ones((S, S))).astype(np.float32)
_G = np.tile(np.eye(C, dtype=np.float32), (S, 1))


def _ln(x, g, b, eps=1e-5):
    m = jnp.mean(x, axis=-1, keepdims=True)
    xc = x - m
    v = jnp.mean(xc * xc, axis=-1, keepdims=True)
    return xc * lax.rsqrt(v + eps) * g + b


def _tab_kernel(x_ref, wq, bq, wk, bk, wv, bv, wo, bo, w1, b1, w2, b2,
                ln1g, ln1b, ln2g, ln2b, tng, tnb, ebig, ones8, fmat, gmat,
                o_ref):
    q = []
    k = []
    vcat_parts = []
    for s_i in range(S):
        xs = x_ref[s_i]
        q.append(jnp.dot(xs, wq[...], preferred_element_type=jnp.float32) + bq[...])
        k.append(jnp.dot(xs, wk[...], preferred_element_type=jnp.float32) + bk[...])
        vcat_parts.append(
            jnp.dot(xs, wv[...], preferred_element_type=jnp.float32) + bv[...])
    vcat = jnp.concatenate(vcat_parts, axis=-1)
    for s_i in range(S):
        qk = jnp.concatenate([q[s_i] * k[t] for t in range(S)], axis=-1)
        scores = jnp.dot(qk, ebig[...], preferred_element_type=jnp.float32)
        es = jnp.exp(scores)
        denom = jnp.dot(es, ones8[...], preferred_element_type=jnp.float32)
        attn = es / denom
        bcast = jnp.dot(attn, fmat[...], preferred_element_type=jnp.float32)
        ctx = jnp.dot(bcast * vcat, gmat[...], preferred_element_type=jnp.float32)
        proj = jnp.dot(ctx, wo[...], preferred_element_type=jnp.float32) + bo[...]
        x1 = _ln(x_ref[s_i] + proj, ln1g[...], ln1b[...])
        ff = jnp.maximum(
            jnp.dot(x1, w1[...], preferred_element_type=jnp.float32) + b1[...], 0.0)
        ff = jnp.dot(ff, w2[...], preferred_element_type=jnp.float32) + b2[...]
        x2 = _ln(x1 + ff, ln2g[...], ln2b[...])
        o_ref[s_i] = _ln(x2, tng[...], tnb[...])


def _tab_branch(x_tab_t, p, B):
    n = x_tab_t.shape[1]
    row = lambda a: a.reshape(1, C)
    args = [
        p['Wq'], row(p['bq']), p['Wk'], row(p['bk']), p['Wv'], row(p['bv']),
        p['Wo'], row(p['bo']), p['W1'], row(p['b1']), p['W2'], row(p['b2']),
        row(p['ln1_g']), row(p['ln1_b']), row(p['ln2_g']), row(p['ln2_b']),
        row(p['tn_g']), row(p['tn_b']),
        jnp.asarray(_E_big), jnp.asarray(_ONES8), jnp.asarray(_F), jnp.asarray(_G),
    ]
    specs = [pl.BlockSpec(a.shape, lambda i: (0, 0)) for a in args]
    return pl.pallas_call(
        _tab_kernel,
        out_shape=jax.ShapeDtypeStruct((S, n, C), jnp.float32),
        grid=(n // B,),
        in_specs=[pl.BlockSpec((S, B, C), lambda i: (0, i, 0))] + specs,
        out_specs=pl.BlockSpec((S, B, C), lambda i: (0, i, 0)),
    )(x_tab_t, *args)


def _mm_bias_kernel(x_ref, w_ref, b_ref, o_ref):
    o_ref[...] = jnp.dot(x_ref[...], w_ref[...],
                         preferred_element_type=jnp.float32) + b_ref[...]


def _mm_bias(x, w, bias, blk):
    n = x.shape[0]
    return pl.pallas_call(
        _mm_bias_kernel,
        out_shape=jax.ShapeDtypeStruct((n, C), jnp.float32),
        grid=(n // blk,),
        in_specs=[
            pl.BlockSpec((blk, x.shape[1]), lambda i: (i, 0)),
            pl.BlockSpec(w.shape, lambda i: (0, 0)),
            pl.BlockSpec((1, C), lambda i: (0, 0)),
        ],
        out_specs=pl.BlockSpec((blk, C), lambda i: (i, 0)),
    )(x, w, bias.reshape(1, C))


def _ab_kernel(x_ref, w1_ref, w2_ref, a_ref, b_ref):
    x = x_ref[...]
    a_ref[...] = jnp.dot(x, w1_ref[...], preferred_element_type=jnp.float32)
    b_ref[...] = jnp.dot(x, w2_ref[...], preferred_element_type=jnp.float32)


def _ab_proj(x_pad, wp1, wp2):
    blk = NP // 4
    return pl.pallas_call(
        _ab_kernel,
        out_shape=[jax.ShapeDtypeStruct((NP, C), jnp.float32)] * 2,
        grid=(4,),
        in_specs=[
            pl.BlockSpec((blk, C), lambda i: (i, 0)),
            pl.BlockSpec((C, C), lambda i: (0, 0)),
            pl.BlockSpec((C, C), lambda i: (0, 0)),
        ],
        out_specs=[pl.BlockSpec((blk, C), lambda i: (i, 0))] * 2,
    )(x_pad, wp1, wp2)


_SC_PARAMS = pltpu.CompilerParams(needs_layout_passes=False,
                                  use_tc_tiling_on_sc=False)


def _sc_mesh():
    return plsc.VectorSubcoreMesh(core_axis_name="c", subcore_axis_name="s")


def _k1_count(dst):
    @functools.partial(
        pl.kernel, mesh=_sc_mesh(), compiler_params=_SC_PARAMS,
        out_type=jax.ShapeDtypeStruct((SLICES, 16), jnp.int32),
        scratch_types=[
            pltpu.VMEM((SCAN_CH,), jnp.int32),
            pltpu.VMEM((16,), jnp.int32),
        ],
    )
    def k(dst_hbm, cnt_hbm, dbuf, cvec_v):
        wid = lax.axis_index("s") * 2 + lax.axis_index("c")
        iot = lax.iota(jnp.int32, 16)
        cvec = jnp.zeros((16,), jnp.int32)
        for ch in range(SLICE_E // SCAN_CH):
            pltpu.sync_copy(
                dst_hbm.at[pl.ds(wid * SLICE_E + ch * SCAN_CH, SCAN_CH)], dbuf)

            def body(i, cv):
                d16 = dbuf[pl.ds(i * 16, 16)]
                oct16 = d16 // OSZ
                for o in range(8):
                    pc = plsc.all_reduce_population_count(oct16 == o)
                    cv = cv + jnp.where(iot == o, pc, 0)
                return cv
            cvec = lax.fori_loop(0, SCAN_CH // 16, body, cvec)
        cvec_v[...] = cvec
        pltpu.sync_copy(cvec_v, cnt_hbm.at[wid])
    return k(dst)


def _k2_bucket(dst, src, c, b_pad, bases):
    @functools.partial(
        pl.kernel, mesh=_sc_mesh(), compiler_params=_SC_PARAMS,
        out_type=[
            jax.ShapeDtypeStruct((8, CAP_T, 16), jnp.float32),
            jax.ShapeDtypeStruct((CAP_T,), jnp.int32),
        ],
        scratch_types=[
            pltpu.VMEM((SCAN_CH,), jnp.int32),
            pltpu.VMEM((SCAN_CH,), jnp.int32),
            pltpu.VMEM((8, 256), jnp.int32),
            pltpu.VMEM((8, 256), jnp.int32),
            pltpu.VMEM((8, 256), jnp.int32),
            pltpu.VMEM((CH, C), jnp.float32),
            pltpu.VMEM((CH, C), jnp.float32),
            pltpu.VMEM((8, CH, 16), jnp.float32),
            pltpu.VMEM((16,), jnp.int32),
            pltpu.SMEM((8,), jnp.int32),
            pltpu.SMEM((8,), jnp.int32),
            pltpu.SemaphoreType.DMA,
            pltpu.SemaphoreType.DMA,
        ],
    )
    def k(dst_hbm, src_hbm, c_hbm, b_hbm, bases_hbm, r_hbm, dl_hbm,
          dbuf, sbuf, ids_st, srcs_st, dsts_st, cbuf, bbuf, rg, bvec_v,
          fill_s, wr_s, sem_c, sem_b):
        wid = lax.axis_index("s") * 2 + lax.axis_index("c")
        iot = lax.iota(jnp.int32, 16)
        pltpu.sync_copy(bases_hbm.at[wid], bvec_v)
        bvec = bvec_v[...]
        for o in range(8):
            fill_s[o] = 0
            wr_s[o] = 0

        def flush(o):
            wb = wr_s[o]
            base = bvec[o]
            pos0 = pl.multiple_of(base + wb, CH)
            pltpu.async_copy(c_hbm.at[ids_st.at[o, pl.ds(0, CH)]], cbuf,
                             sem_c).wait()
            pltpu.async_copy(b_hbm.at[srcs_st.at[o, pl.ds(0, CH)]], bbuf,
                             sem_b).wait()

            def comb(j4, _):
                for u in range(4):
                    j = j4 * 4 + u
                    for g in range(8):
                        rg[g, j, :] = (cbuf[j, pl.ds(g * 16, 16)]
                                       + bbuf[j, pl.ds(g * 16, 16)])
                return 0
            lax.fori_loop(0, CH // 4, comb, 0)
            for g in range(8):
                pltpu.sync_copy(rg.at[g], r_hbm.at[g, pl.ds(pos0, CH)])
            pltpu.sync_copy(dsts_st.at[o, pl.ds(0, CH)],
                            dl_hbm.at[pl.ds(pos0, CH)])
            wr_s[o] = wb + CH

        def scan_chunk(ch, _):
            ebase = wid * SLICE_E + ch * SCAN_CH
            pltpu.sync_copy(dst_hbm.at[pl.ds(ebase, SCAN_CH)], dbuf)
            pltpu.sync_copy(src_hbm.at[pl.ds(ebase, SCAN_CH)], sbuf)

            def body(i, _):
                d16 = dbuf[pl.ds(i * 16, 16)]
                s16 = sbuf[pl.ds(i * 16, 16)]
                id16 = iot + (ebase + i * 16)
                oct16 = d16 // OSZ
                for o in range(8):
                    msk = oct16 == o
                    pc = plsc.all_reduce_population_count(msk)
                    f = fill_s[o]
                    plsc.store_compressed(ids_st.at[o, pl.ds(f, 16)], id16,
                                          mask=msk)
                    plsc.store_compressed(srcs_st.at[o, pl.ds(f, 16)], s16,
                                          mask=msk)
                    plsc.store_compressed(dsts_st.at[o, pl.ds(f, 16)], d16,
                                          mask=msk)
                    fill_s[o] = f + pc[0]
                for o in range(8):
                    f = fill_s[o]

                    @pl.when(f >= CH)
                    def _():
                        flush(o)
                        t0 = ids_st[o, pl.ds(CH, 16)]
                        t1 = srcs_st[o, pl.ds(CH, 16)]
                        t2 = dsts_st[o, pl.ds(CH, 16)]
                        ids_st[o, pl.ds(0, 16)] = t0
                        srcs_st[o, pl.ds(0, 16)] = t1
                        dsts_st[o, pl.ds(0, 16)] = t2
                        fill_s[o] = f - CH
                return 0
            lax.fori_loop(0, SCAN_CH // 16, body, 0)
            return 0
        lax.fori_loop(0, SLICE_E // SCAN_CH, scan_chunk, 0)

        zid = jnp.zeros((16,), jnp.int32)
        for o in range(8):
            f = fill_s[o]

            @pl.when(f > 0)
            def _():
                sent = jnp.full((16,), o * OSZ + OSZ, jnp.int32)
                for j in range(CH // 16):
                    ids_st[o, pl.ds(f + j * 16, 16)] = zid
                    srcs_st[o, pl.ds(f + j * 16, 16)] = zid
                    dsts_st[o, pl.ds(f + j * 16, 16)] = sent
                flush(o)
    return k(dst, src, c, b_pad, bases)


def _k3_reduce(r_sc, dlist, bases, padded):
    @functools.partial(
        pl.kernel, mesh=_sc_mesh(), compiler_params=_SC_PARAMS,
        out_type=[
            jax.ShapeDtypeStruct((8, 8, ACC_R, 16), jnp.float32),
            jax.ShapeDtypeStruct((8, 8, ACC_R, 16), jnp.float32),
            jax.ShapeDtypeStruct((8, 8, ACC_R, 16), jnp.float32),
            jax.ShapeDtypeStruct((8, 8, ACC_R, 16), jnp.float32),
            jax.ShapeDtypeStruct((8, 1264), jnp.float32),
        ],
        scratch_types=[
            pltpu.VMEM((ACC_R, 16), jnp.float32),
            pltpu.VMEM((ACC_R, 16), jnp.float32),
            pltpu.VMEM((ACC_R, 16), jnp.float32),
            pltpu.VMEM((ACC_R, 16), jnp.float32),
            pltpu.VMEM((ACC_R, 16), jnp.float32),
            pltpu.VMEM((ACC_R, 16), jnp.float32),
            pltpu.VMEM((1264,), jnp.float32),
            pltpu.VMEM((2 * CH, 16), jnp.float32),
            pltpu.VMEM((2 * CH,), jnp.int32),
            pltpu.VMEM((SLICES, 16), jnp.int32),
            pltpu.VMEM((SLICES, 16), jnp.int32),
        ],
    )
    def k(r_hbm, dl_hbm, bases_hbm, pad_hbm, s1_hbm, s2_hbm, mx_hbm, mn_hbm,
          cnt_hbm, a1, a2, aM0, aM1, am0, am1, acn, rbuf, dbuf, bas_v, pad_v):
        wid = lax.axis_index("s") * 2 + lax.axis_index("c")
        g = wid % 8
        slot = wid // 8
        iot = lax.iota(jnp.int32, 16)
        ones = jnp.ones((16,), jnp.float32)
        m0 = iot == 0
        pltpu.sync_copy(bases_hbm, bas_v)
        pltpu.sync_copy(pad_hbm, pad_v)
        for rnd in range(2):
            o = slot + 4 * rnd
            obase = o * OSZ

            def init(i, _):
                z = jnp.zeros((16,), jnp.float32)
                a1[i, :] = z
                a2[i, :] = z
                aM0[i, :] = jnp.full((16,), NEG, jnp.float32)
                aM1[i, :] = jnp.full((16,), NEG, jnp.float32)
                am0[i, :] = jnp.full((16,), POS, jnp.float32)
                am1[i, :] = jnp.full((16,), POS, jnp.float32)
                return 0
            lax.fori_loop(0, ACC_R, init, 0)

            def initc(i, _):
                acn[pl.ds(i * 16, 16)] = jnp.zeros((16,), jnp.float32)
                return 0
            lax.fori_loop(0, 1264 // 16, initc, 0)

            def slice_body(p, _):
                brow = bas_v[p, :]
                prow = pad_v[p, :]
                base = brow[jnp.full((16,), o, jnp.int32)][0]
                npad = prow[jnp.full((16,), o, jnp.int32)][0]

                def vstep(v, _):
                    d16 = dbuf[pl.ds(v * 16, 16)] - obase
                    for j in range(16):
                        db = d16[jnp.full((16,), j, jnp.int32)]
                        r = rbuf[v * 16 + j, :]
                        plsc.addupdate_scatter(a1, [db, iot], r)
                        plsc.addupdate_scatter(a2, [db, iot], r * r)
                        aM = aM0 if j % 2 == 0 else aM1
                        am = am0 if j % 2 == 0 else am1
                        om = plsc.load_gather(aM, [db, iot])
                        plsc.store_scatter(aM, [db, iot], jnp.maximum(om, r))
                        on = plsc.load_gather(am, [db, iot])
                        plsc.store_scatter(am, [db, iot], jnp.minimum(on, r))
                        plsc.addupdate_scatter(acn, [db], ones, mask=m0)
                    return 0

                def chunk2(ci, _):
                    off = pl.multiple_of(base + ci * 2 * CH, CH)
                    pltpu.sync_copy(r_hbm.at[g, pl.ds(off, 2 * CH)], rbuf)
                    pltpu.sync_copy(dl_hbm.at[pl.ds(off, 2 * CH)], dbuf)
                    lax.fori_loop(0, 2 * CH // 16, vstep, 0)
                    return 0
                nbig = npad // (2 * CH)
                lax.fori_loop(0, nbig, chunk2, 0)

                @pl.when(npad % (2 * CH) != 0)
                def _():
                    off = pl.multiple_of(base + nbig * 2 * CH, CH)
                    pltpu.sync_copy(r_hbm.at[g, pl.ds(off, CH)],
                                    rbuf.at[pl.ds(0, CH)])
                    pltpu.sync_copy(dl_hbm.at[pl.ds(off, CH)],
                                    dbuf.at[pl.ds(0, CH)])
                    lax.fori_loop(0, CH // 16, vstep, 0)
                return 0
            lax.fori_loop(0, SLICES, slice_body, 0)

            def merge(i, _):
                aM0[i, :] = jnp.maximum(aM0[i, :], aM1[i, :])
                am0[i, :] = jnp.minimum(am0[i, :], am1[i, :])
                return 0
            lax.fori_loop(0, ACC_R, merge, 0)
            pltpu.sync_copy(a1, s1_hbm.at[g, o])
            pltpu.sync_copy(a2, s2_hbm.at[g, o])
            pltpu.sync_copy(aM0, mx_hbm.at[g, o])
            pltpu.sync_copy(am0, mn_hbm.at[g, o])

            @pl.when(g == 0)
            def _():
                pltpu.sync_copy(acn, cnt_hbm.at[o])
    return k(r_sc, dlist, bases, padded)


def _node_kernel(xg_ref, a_ref, cb_ref, s1_ref, s2_ref, mx_ref, mn_ref,
                 wpost_ref, bpost_ref, wlin_ref, blin_ref,
                 out_ref, bs_ref, bq_ref):
    i = pl.program_id(0)
    cnt = cb_ref[...]
    cnt1 = jnp.maximum(cnt, 1.0)
    pos = cnt > 0.0
    inv = 1.0 / cnt1
    a = a_ref[...]
    s1 = s1_ref[...] * inv
    mean = jnp.where(pos, a + s1, 0.0)
    mx = jnp.where(pos, a + mx_ref[...], 0.0)
    mn = jnp.where(pos, a + mn_ref[...], 0.0)
    var = s2_ref[...] * inv - s1 * s1
    std = jnp.sqrt(jnp.maximum(var, 0.0) + 1e-5)
    lg = jnp.log(cnt1 + 1.0)
    amp = lg * (1.0 / AVG_LOG)
    att = AVG_LOG / lg
    wp = wpost_ref
    acc = jnp.dot(xg_ref[...], wp[pl.ds(0, C), :],
                  preferred_element_type=jnp.float32)
    aggs = [mean, mx, mn, std]
    for sc_i, scl in enumerate([None, amp, att]):
        for a_i in range(4):
            blkidx = 1 + sc_i * 4 + a_i
            x = aggs[a_i] if scl is None else aggs[a_i] * scl
            acc = acc + jnp.dot(x, wp[pl.ds(blkidx * C, C), :],
                                preferred_element_type=jnp.float32)
    acc = acc + bpost_ref[...]
    out = jnp.dot(acc, wlin_ref[...],
                  preferred_element_type=jnp.float32) + blin_ref[...]
    out_ref[...] = out

    @pl.when(i == 0)
    def _():
        bs_ref[...] = jnp.zeros_like(bs_ref)
        bq_ref[...] = jnp.zeros_like(bq_ref)
    bs_ref[...] += jnp.sum(out, axis=0, keepdims=True)
    bq_ref[...] += jnp.sum(out * out, axis=0, keepdims=True)


def _node_stage(x_gnn, a, cnt_b, s1, s2, mx, mn, p):
    blk = 2000
    cspec = lambda shp: pl.BlockSpec(shp, lambda i: (0, 0))
    bspec = pl.BlockSpec((blk, C), lambda i: (i, 0))
    return pl.pallas_call(
        _node_kernel,
        out_shape=[
            jax.ShapeDtypeStruct((N, C), jnp.float32),
            jax.ShapeDtypeStruct((1, C), jnp.float32),
            jax.ShapeDtypeStruct((1, C), jnp.float32),
        ],
        grid=(N // blk,),
        in_specs=[bspec] * 7 + [
            cspec((13 * C, C)), cspec((1, C)), cspec((C, C)), cspec((1, C)),
        ],
        out_specs=[bspec, cspec((1, C)), cspec((1, C))],
    )(x_gnn, a, cnt_b, s1, s2, mx, mn,
      p['Wpost'], p['bpost'].reshape(1, C), p['Wlin'], p['blin'].reshape(1, C))


def _bn_combine_kernel(out_ref, xg_ref, bm_ref, bv_ref, g_ref, b_ref, o_ref):
    y = (out_ref[...] - bm_ref[...]) / jnp.sqrt(bv_ref[...] + 1e-5) \
        * g_ref[...] + b_ref[...]
    o_ref[...] = (xg_ref[...] + jnp.maximum(y, 0.0)) / 2.0


def kernel(x_tab, x_gnn, edge_index, edge_attr, params):
    p = params

    x_tab_t = x_tab.transpose(1, 0, 2)
    x_tab_out = _tab_branch(x_tab_t, p, 1000).transpose(1, 0, 2)

    src = edge_index[0]
    dst = edge_index[1]
    Wp1, Wp2, Wp3 = p['Wpre'][:C], p['Wpre'][C:2 * C], p['Wpre'][2 * C:]

    x_pad = jnp.concatenate([x_gnn, jnp.zeros((NP - N, C), jnp.float32)], 0)
    a_pad, b_pad = _ab_proj(x_pad, Wp1, Wp2)
    a = a_pad[:N]
    c = _mm_bias(edge_attr, p['We'] @ Wp3, p['be'] @ Wp3 + p['bpre'], 2000)

    counts = _k1_count(dst)[:, :8]
    padded = ((counts + (CH - 1)) // CH) * CH
    bases_flat = jnp.concatenate(
        [jnp.zeros((1,), jnp.int32), jnp.cumsum(padded.reshape(-1))[:-1]])
    bases = bases_flat.reshape(SLICES, 8)
    bases16 = jnp.concatenate(
        [bases, jnp.zeros((SLICES, 8), jnp.int32)], axis=1)
    padded16 = jnp.concatenate(
        [padded, jnp.zeros((SLICES, 8), jnp.int32)], axis=1)

    r_sc, dlist = _k2_bucket(dst, src, c, b_pad, bases16)
    s1_4, s2_4, mx_4, mn_4, cnt_2 = _k3_reduce(r_sc, dlist, bases16, padded16)

    def _unacc(x4):
        return (x4[:, :, :OSZ, :].transpose(1, 2, 0, 3)
                .reshape(NP, C)[:N])
    s1 = _unacc(s1_4)
    s2 = _unacc(s2_4)
    mxa = _unacc(mx_4)
    mna = _unacc(mn_4)
    cnt = cnt_2[:, :OSZ].reshape(NP)[:N]
    cnt_b = jnp.broadcast_to(cnt[:, None], (N, C))

    out_pre, bs, bq = _node_stage(x_gnn, a, cnt_b, s1, s2, mxa, mna, p)
    bm = bs / float(N)
    bv = bq / float(N) - bm * bm

    x_gnn_out = pl.pallas_call(
        _bn_combine_kernel,
        out_shape=jax.ShapeDtypeStruct((N, C), jnp.float32),
        grid=(N // 2000,),
        in_specs=[
            pl.BlockSpec((2000, C), lambda i: (i, 0)),
            pl.BlockSpec((2000, C), lambda i: (i, 0)),
            pl.BlockSpec((1, C), lambda i: (0, 0)),
            pl.BlockSpec((1, C), lambda i: (0, 0)),
            pl.BlockSpec((1, C), lambda i: (0, 0)),
            pl.BlockSpec((1, C), lambda i: (0, 0)),
        ],
        out_specs=pl.BlockSpec((2000, C), lambda i: (i, 0)),
    )(out_pre, x_gnn, bm, bv, p['bn_g'].reshape(1, C), p['bn_b'].reshape(1, C))

    return (x_tab_out, x_gnn_out, edge_attr)
